# Initial kernel scaffold; baseline (speedup 1.0000x reference)
import jax
import jax.numpy as jnp
from jax.experimental import pallas as pl


def kernel(boxes, scores, class_idxs):
    raise NotImplementedError("write your pallas kernel here")


# speedup vs baseline: 68.9620x; 68.9620x over previous
import jax
import jax.numpy as jnp
from jax.experimental import pallas as pl

_IOU_T = 0.5
_MAX_DET = 100
_B = 256


def _nms_body(rows_ref, cols_ref, alive_ref):
    npad = cols_ref.shape[1]
    nb = npad // _B
    alive_ref[...] = jnp.ones((1, npad), jnp.float32)
    cx1 = cols_ref[0:1, :]
    cy1 = cols_ref[1:2, :]
    cx2 = cols_ref[2:3, :]
    cy2 = cols_ref[3:4, :]
    car = cols_ref[4:5, :]
    colg = jax.lax.broadcasted_iota(jnp.int32, (_B, npad), 1)
    coll = jax.lax.broadcasted_iota(jnp.int32, (_B, _B), 1)
    rowl = jax.lax.broadcasted_iota(jnp.int32, (_B, _B), 0)

    def block_step(i, carry):
        base = i * _B
        rb = rows_ref[pl.ds(base, _B), :]
        rx1 = rb[:, 0:1]
        ry1 = rb[:, 1:2]
        rx2 = rb[:, 2:3]
        ry2 = rb[:, 3:4]
        rar = rb[:, 4:5]

        iw = jnp.maximum(jnp.minimum(rx2, cx2) - jnp.maximum(rx1, cx1), 0.0)
        ih = jnp.maximum(jnp.minimum(ry2, cy2) - jnp.maximum(ry1, cy1), 0.0)
        inter = iw * ih
        iou = inter / (rar + car - inter + 1e-9)
        rowg = base + jax.lax.broadcasted_iota(jnp.int32, (_B, npad), 0)
        over = jnp.where((iou > _IOU_T) & (colg > rowg), 1.0, 0.0)

        cb = cols_ref[:, pl.ds(base, _B)]
        dw = jnp.maximum(jnp.minimum(rx2, cb[2:3, :]) - jnp.maximum(rx1, cb[0:1, :]), 0.0)
        dh = jnp.maximum(jnp.minimum(ry2, cb[3:4, :]) - jnp.maximum(ry1, cb[1:2, :]), 0.0)
        dinter = dw * dh
        diou = dinter / (rar + cb[4:5, :] - dinter + 1e-9)
        oblk = jnp.where((diou > _IOU_T) & (coll > rowl), 1.0, 0.0)

        inc = alive_ref[:, pl.ds(base, _B)]

        def w_cond(c):
            return c[1]

        def w_body(c):
            a, _ = c
            sup = jax.lax.dot_general(
                a, oblk, (((1,), (0,)), ((), ())),
                preferred_element_type=jnp.float32)
            a_new = jnp.where(sup > 0.0, 0.0, inc)
            return a_new, jnp.any(a_new != a)

        a_fin, _ = jax.lax.while_loop(w_cond, w_body, (inc, jnp.bool_(True)))

        supall = jax.lax.dot_general(
            a_fin, over, (((1,), (0,)), ((), ())),
            preferred_element_type=jnp.float32)
        alive_ref[...] = jnp.where(supall > 0.0, 0.0, alive_ref[...])
        return carry

    jax.lax.fori_loop(0, nb, block_step, 0)


@jax.jit
def kernel(boxes, scores, class_idxs):
    n = boxes.shape[0]
    npad = ((n + _B - 1) // _B) * _B
    max_coord = jnp.max(boxes) + 1.0
    offsets = class_idxs.astype(boxes.dtype) * max_coord
    boxes_for_nms = boxes + offsets[:, None]

    order = jnp.argsort(-scores)
    bs = boxes_for_nms[order]
    area = (bs[:, 2] - bs[:, 0]) * (bs[:, 3] - bs[:, 1])
    feat = jnp.concatenate(
        [bs, area[:, None], jnp.zeros((n, 3), jnp.float32)], axis=1)
    featp = jnp.zeros((npad, 8), jnp.float32).at[:n].set(feat)

    alive = pl.pallas_call(
        _nms_body,
        out_shape=jax.ShapeDtypeStruct((1, npad), jnp.float32),
    )(featp, featp.T)

    keep = alive[0, :n] > 0.5
    kept_scores = jnp.where(keep, scores[order], -jnp.inf)
    _, topk_idx = jax.lax.top_k(kept_scores, _MAX_DET)
    final_idx = order[topk_idx]
    return boxes[final_idx], scores[final_idx], class_idxs[final_idx]
